# Initial kernel scaffold; baseline (speedup 1.0000x reference)
import jax
import jax.numpy as jnp
from jax.experimental import pallas as pl


def kernel(x, edge_index, edge_type, W, W_root, bias):
    raise NotImplementedError("write your pallas kernel here")


# speedup vs baseline: 10.3796x; 10.3796x over previous
import functools

import jax
import jax.numpy as jnp
from jax import lax
from jax.experimental import pallas as pl
from jax.experimental.pallas import tpu as pltpu
from jax.experimental.pallas import tpu_sc as plsc

L = 16


def _tc_xw(x, W):
    N, D = x.shape
    R = W.shape[0]
    BN = 1000

    def body(x_ref, w_ref, o_ref):
        xb = x_ref[...]
        for r in range(R):
            o_ref[r] = jnp.dot(xb, w_ref[r], preferred_element_type=jnp.float32)

    return pl.pallas_call(
        body,
        grid=(N // BN,),
        in_specs=[
            pl.BlockSpec((BN, D), lambda i: (i, 0)),
            pl.BlockSpec((R, D, D), lambda i: (0, 0, 0)),
        ],
        out_specs=pl.BlockSpec((R, BN, D), lambda i: (0, i, 0)),
        out_shape=jax.ShapeDtypeStruct((R, N, D), jnp.float32),
    )(x, W)


def _tc_out(x, W_root, bias, acc):
    N, D = x.shape
    BN = 1000

    def body(x_ref, wr_ref, b_ref, a_ref, o_ref):
        o_ref[...] = (
            jnp.dot(x_ref[...], wr_ref[...], preferred_element_type=jnp.float32)
            + b_ref[...]
            + a_ref[0]
            + a_ref[1]
        )

    return pl.pallas_call(
        body,
        grid=(N // BN,),
        in_specs=[
            pl.BlockSpec((BN, D), lambda i: (i, 0)),
            pl.BlockSpec((D, D), lambda i: (0, 0)),
            pl.BlockSpec((1, D), lambda i: (0, 0)),
            pl.BlockSpec((2, BN, D), lambda i: (0, i, 0)),
        ],
        out_specs=pl.BlockSpec((BN, D), lambda i: (i, 0)),
        out_shape=jax.ShapeDtypeStruct((N, D), jnp.float32),
    )(x, W_root, bias.reshape(1, D), acc)


def _sc_count(E, N, R, NC, NS, CNT):
    EPT = E // (NC * NS)
    CH = 80
    NCHUNK = EPT // CH
    SL = CNT // NS
    mesh = plsc.VectorSubcoreMesh(core_axis_name="c", subcore_axis_name="s")

    @functools.partial(
        pl.kernel,
        mesh=mesh,
        out_type=jax.ShapeDtypeStruct((NC * CNT,), jnp.float32),
        scratch_types=[
            pltpu.VMEM((CH,), jnp.int32),
            pltpu.VMEM((CH,), jnp.int32),
            pltpu.VMEM((CH,), jnp.int32),
            pltpu.VMEM((CH,), jnp.float32),
            pltpu.VMEM((SL,), jnp.float32),
            pltpu.VMEM_SHARED((CNT,), jnp.float32),
        ],
        compiler_params=pltpu.CompilerParams(needs_layout_passes=False),
    )
    def k(et_hbm, dst_hbm, out_hbm, et_c, dst_c, cidx, ones, zbuf, cnt_sp):
        cid = lax.axis_index("c")
        sid = lax.axis_index("s")
        wid = cid * NS + sid
        zeros16 = jnp.zeros((L,), jnp.float32)
        ones16 = jnp.full((L,), 1.0, jnp.float32)

        def zloop(i, _):
            zbuf[pl.ds(i * L, L)] = zeros16
            return _

        lax.fori_loop(0, SL // L, zloop, None)
        for q in range(CH // L):
            ones[pl.ds(q * L, L)] = ones16
        pltpu.sync_copy(zbuf, cnt_sp.at[pl.ds(sid * SL, SL)])
        plsc.subcore_barrier()

        def chunk(ci, _):
            base = wid * EPT + ci * CH
            pltpu.sync_copy(et_hbm.at[pl.ds(base, CH)], et_c)
            pltpu.sync_copy(dst_hbm.at[pl.ds(base, CH)], dst_c)
            for q in range(CH // L):
                c16 = et_c[pl.ds(q * L, L)] * N + dst_c[pl.ds(q * L, L)]
                cidx[pl.ds(q * L, L)] = c16
            pltpu.sync_copy(ones, cnt_sp.at[cidx], add=True)
            return _

        lax.fori_loop(0, NCHUNK, chunk, None)
        plsc.subcore_barrier()
        pltpu.sync_copy(
            cnt_sp.at[pl.ds(sid * SL, SL)],
            out_hbm.at[pl.ds(cid * CNT + sid * SL, SL)],
        )

    return k


def _sc_scales(E, N, R, NC, NS, CNT):
    EPT = E // (NC * NS)
    CH = 80
    NCHUNK = EPT // CH
    SL = CNT // L
    mesh = plsc.VectorSubcoreMesh(core_axis_name="c", subcore_axis_name="s")

    @functools.partial(
        pl.kernel,
        mesh=mesh,
        out_type=jax.ShapeDtypeStruct((E,), jnp.float32),
        scratch_types=[
            pltpu.VMEM((CNT,), jnp.float32),
            pltpu.VMEM((SL,), jnp.float32),
            pltpu.VMEM((SL,), jnp.float32),
            pltpu.VMEM((CH,), jnp.int32),
            pltpu.VMEM((CH,), jnp.int32),
            pltpu.VMEM((CH,), jnp.float32),
        ],
        compiler_params=pltpu.CompilerParams(needs_layout_passes=False),
    )
    def k(et_hbm, dst_hbm, cnt_hbm, out_hbm, inv, t0, t1, dst_c, et_c, s_val):
        cid = lax.axis_index("c")
        sid = lax.axis_index("s")
        wid = cid * NS + sid
        one16 = jnp.full((L,), 1.0, jnp.float32)

        for blk in range(L):
            pltpu.sync_copy(cnt_hbm.at[pl.ds(blk * SL, SL)], t0)
            pltpu.sync_copy(cnt_hbm.at[pl.ds(CNT + blk * SL, SL)], t1)

            def invloop(i, _, blk=blk):
                c = t0[pl.ds(i * L, L)] + t1[pl.ds(i * L, L)]
                inv[pl.ds(blk * SL + i * L, L)] = one16 / jnp.maximum(c, one16)
                return _

            lax.fori_loop(0, SL // L, invloop, None)

        def chunk(ci, _):
            base = wid * EPT + ci * CH
            pltpu.sync_copy(et_hbm.at[pl.ds(base, CH)], et_c)
            pltpu.sync_copy(dst_hbm.at[pl.ds(base, CH)], dst_c)
            for q in range(CH // L):
                c16 = et_c[pl.ds(q * L, L)] * N + dst_c[pl.ds(q * L, L)]
                s_val[pl.ds(q * L, L)] = plsc.load_gather(inv, [c16])
            pltpu.sync_copy(s_val, out_hbm.at[pl.ds(base, CH)])
            return _

        lax.fori_loop(0, NCHUNK, chunk, None)

    return k


def _sc_main(E, N, R, NC, NS):
    EPT = E // (NC * NS)
    CH = 80
    NCHUNK = EPT // CH
    RPT = (N // NS) // 8 * 8
    REM = N - RPT * NS
    mesh = plsc.VectorSubcoreMesh(core_axis_name="c", subcore_axis_name="s")

    @functools.partial(
        pl.kernel,
        mesh=mesh,
        out_type=jax.ShapeDtypeStruct((NC, N, 128), jnp.float32),
        scratch_types=[
            pltpu.VMEM((CH, 128), jnp.float32),
            pltpu.VMEM((CH,), jnp.int32),
            pltpu.VMEM((CH,), jnp.int32),
            pltpu.VMEM((CH,), jnp.int32),
            pltpu.VMEM((CH,), jnp.int32),
            pltpu.VMEM((CH,), jnp.float32),
            pltpu.VMEM_SHARED((N, 128), jnp.float32),
            pltpu.SemaphoreType.DMA,
        ],
        compiler_params=pltpu.CompilerParams(needs_layout_passes=False),
    )
    def k(
        xw_hbm,
        src_hbm,
        dst_hbm,
        et_hbm,
        s_hbm,
        out_hbm,
        rows,
        src_c,
        dst_c,
        et_c,
        g_idx,
        s_val,
        acc_sp,
        sem,
    ):
        cid = lax.axis_index("c")
        sid = lax.axis_index("s")
        wid = cid * NS + sid
        zeros16 = jnp.zeros((L,), jnp.float32)

        def zrow(i, _):
            for j in range(8):
                rows[i, pl.ds(j * L, L)] = zeros16
            return _

        lax.fori_loop(0, CH, zrow, None)
        nfull = RPT // CH
        for t in range(nfull):
            pltpu.sync_copy(rows, acc_sp.at[pl.ds(sid * RPT + t * CH, CH)])
        rem = RPT - nfull * CH
        if rem:
            pltpu.sync_copy(
                rows.at[pl.ds(0, rem)],
                acc_sp.at[pl.ds(sid * RPT + nfull * CH, rem)],
            )
        if REM:

            @pl.when(sid == NS - 1)
            def _():
                pltpu.sync_copy(
                    rows.at[pl.ds(0, REM)], acc_sp.at[pl.ds(RPT * NS, REM)]
                )

        plsc.subcore_barrier()

        def chunk(ci, _):
            base = wid * EPT + ci * CH
            pltpu.sync_copy(src_hbm.at[pl.ds(base, CH)], src_c)
            pltpu.sync_copy(dst_hbm.at[pl.ds(base, CH)], dst_c)
            pltpu.sync_copy(et_hbm.at[pl.ds(base, CH)], et_c)
            pltpu.sync_copy(s_hbm.at[pl.ds(base, CH)], s_val)
            for q in range(CH // L):
                g_idx[pl.ds(q * L, L)] = (
                    et_c[pl.ds(q * L, L)] * N + src_c[pl.ds(q * L, L)]
                )
            pltpu.async_copy(xw_hbm.at[g_idx], rows, sem).wait()

            def scale(kk, _):
                sk = plsc.load_gather(s_val, [jnp.full((L,), kk, jnp.int32)])
                for j in range(8):
                    rows[kk, pl.ds(j * L, L)] = rows[kk, pl.ds(j * L, L)] * sk
                return _

            lax.fori_loop(0, CH, scale, None)
            pltpu.sync_copy(rows, acc_sp.at[dst_c], add=True)
            return _

        lax.fori_loop(0, NCHUNK, chunk, None)
        plsc.subcore_barrier()

        pltpu.sync_copy(
            acc_sp.at[pl.ds(sid * RPT, RPT)],
            out_hbm.at[cid, pl.ds(sid * RPT, RPT)],
        )
        if REM:

            @pl.when(sid == NS - 1)
            def _():
                pltpu.sync_copy(
                    acc_sp.at[pl.ds(RPT * NS, REM)],
                    out_hbm.at[cid, pl.ds(RPT * NS, REM)],
                )

    return k


def kernel(x, edge_index, edge_type, W, W_root, bias):
    N, D = x.shape
    R = W.shape[0]
    E = edge_type.shape[0]
    NC, NS = 2, 16
    CNT = 81920
    assert D == 128 and R * N <= CNT and E % (NC * NS * 80) == 0 and N % NS == 0

    src = edge_index[0]
    dst = edge_index[1]

    xw = _tc_xw(x, W).reshape(R * N, D)
    cnt = _sc_count(E, N, R, NC, NS, CNT)(edge_type, dst)
    s = _sc_scales(E, N, R, NC, NS, CNT)(edge_type, dst, cnt)
    acc = _sc_main(E, N, R, NC, NS)(xw, src, dst, edge_type, s)
    return _tc_out(x, W_root, bias, acc)


# speedup vs baseline: 39.7119x; 3.8259x over previous
import functools

import jax
import jax.numpy as jnp
from jax import lax
from jax.experimental import pallas as pl
from jax.experimental.pallas import tpu as pltpu
from jax.experimental.pallas import tpu_sc as plsc

L = 16


def _tc_xw(x, W):
    N, D = x.shape
    R = W.shape[0]
    BN = 1000

    def body(x_ref, w_ref, o_ref):
        xb = x_ref[...]
        for r in range(R):
            o_ref[r] = jnp.dot(xb, w_ref[r], preferred_element_type=jnp.float32)

    return pl.pallas_call(
        body,
        grid=(N // BN,),
        in_specs=[
            pl.BlockSpec((BN, D), lambda i: (i, 0)),
            pl.BlockSpec((R, D, D), lambda i: (0, 0, 0)),
        ],
        out_specs=pl.BlockSpec((R, BN, D), lambda i: (0, i, 0)),
        out_shape=jax.ShapeDtypeStruct((R, N, D), jnp.float32),
    )(x, W)


def _tc_out(x, W_root, bias, acc):
    N, D = x.shape
    BN = 1000

    def body(x_ref, wr_ref, b_ref, a_ref, o_ref):
        o_ref[...] = (
            jnp.dot(x_ref[...], wr_ref[...], preferred_element_type=jnp.float32)
            + b_ref[...]
            + a_ref[0]
            + a_ref[1]
        )

    return pl.pallas_call(
        body,
        grid=(N // BN,),
        in_specs=[
            pl.BlockSpec((BN, D), lambda i: (i, 0)),
            pl.BlockSpec((D, D), lambda i: (0, 0)),
            pl.BlockSpec((1, D), lambda i: (0, 0)),
            pl.BlockSpec((2, BN, D), lambda i: (0, i, 0)),
        ],
        out_specs=pl.BlockSpec((BN, D), lambda i: (i, 0)),
        out_shape=jax.ShapeDtypeStruct((N, D), jnp.float32),
    )(x, W_root, bias.reshape(1, D), acc)


def _sc_prep(E, N, R, NC, NS, CNT):
    CPT = E // NS
    CB = 4000
    NB = CPT // CB
    CH = 80
    EPT = E // (NC * NS)
    HS = 4992
    TAIL = EPT - 2 * HS
    SLAB = CNT // NS
    mesh = plsc.VectorSubcoreMesh(core_axis_name="c", subcore_axis_name="s")

    @functools.partial(
        pl.kernel,
        mesh=mesh,
        out_type=(
            jax.ShapeDtypeStruct((E,), jnp.float32),
            jax.ShapeDtypeStruct((E,), jnp.int32),
        ),
        scratch_types=[
            pltpu.VMEM((CNT,), jnp.float32),
            pltpu.VMEM((SLAB,), jnp.float32),
            pltpu.VMEM((HS,), jnp.int32),
            pltpu.VMEM((HS,), jnp.int32),
            pltpu.VMEM((HS,), jnp.int32),
            pltpu.VMEM((HS,), jnp.float32),
            pltpu.VMEM((HS,), jnp.int32),
            pltpu.VMEM((2, CH), jnp.int32),
            pltpu.VMEM((CH,), jnp.float32),
            pltpu.VMEM_SHARED((CNT,), jnp.float32),
            pltpu.SemaphoreType.DMA,
            pltpu.SemaphoreType.DMA,
        ],
        compiler_params=pltpu.CompilerParams(needs_layout_passes=False),
    )
    def k(
        et_hbm,
        dst_hbm,
        src_hbm,
        s_hbm,
        g_hbm,
        inv,
        tmp,
        b_et,
        b_dst,
        b_src,
        s_out,
        g_out,
        cidx,
        ones,
        cnt_sp,
        ssem0,
        ssem1,
    ):
        cid = lax.axis_index("c")
        sid = lax.axis_index("s")
        wid = cid * NS + sid
        zeros16 = jnp.zeros((L,), jnp.float32)
        one16 = jnp.full((L,), 1.0, jnp.float32)
        ssem = (ssem0, ssem1)

        def z(i, _):
            tmp[pl.ds(i * L, L)] = zeros16
            return _

        lax.fori_loop(0, SLAB // L, z, None)
        for q in range(CH // L):
            ones[pl.ds(q * L, L)] = one16
        pltpu.sync_copy(tmp, cnt_sp.at[pl.ds(sid * SLAB, SLAB)])
        plsc.subcore_barrier()

        cbase = sid * CPT

        def count_chunk(ch, b, guard):
            drain = lambda: pltpu.make_async_copy(
                ones, cnt_sp.at[cidx.at[b]], ssem[b]
            ).wait()
            if guard:
                pl.when(ch >= 2)(drain)
            else:
                drain()
            for q in range(CH // L):
                o = ch * CH + q * L
                c16 = b_et[pl.ds(o, L)] * N + b_dst[pl.ds(o, L)]
                cidx[b, pl.ds(q * L, L)] = c16
            pltpu.async_copy(ones, cnt_sp.at[cidx.at[b]], ssem[b], add=True)

        for blk in range(NB):
            pltpu.sync_copy(
                et_hbm.at[pl.ds(cbase + blk * CB, CB)], b_et.at[pl.ds(0, CB)]
            )
            pltpu.sync_copy(
                dst_hbm.at[pl.ds(cbase + blk * CB, CB)], b_dst.at[pl.ds(0, CB)]
            )

            @pl.loop(0, CB // CH, step=2)
            def _(ch):
                for b in range(2):
                    count_chunk(ch + b, b, blk == 0)

        for b in range(2):
            pltpu.make_async_copy(ones, cnt_sp.at[cidx.at[b]], ssem[b]).wait()
        plsc.subcore_barrier()

        pltpu.sync_copy(cnt_sp.at[pl.ds(sid * SLAB, SLAB)], tmp)

        def invert(i, _):
            c16 = tmp[pl.ds(i * L, L)]
            tmp[pl.ds(i * L, L)] = one16 / jnp.maximum(c16, one16)
            return _

        lax.fori_loop(0, SLAB // L, invert, None)
        pltpu.sync_copy(tmp, cnt_sp.at[pl.ds(sid * SLAB, SLAB)])
        plsc.subcore_barrier()
        pltpu.sync_copy(cnt_sp, inv)

        obase = wid * EPT

        def emit(nv, hb):
            def body(i, _):
                et16 = b_et[pl.ds(i * L, L)]
                c16 = et16 * N + b_dst[pl.ds(i * L, L)]
                s_out[pl.ds(i * L, L)] = plsc.load_gather(inv, [c16])
                g_out[pl.ds(i * L, L)] = et16 * N + b_src[pl.ds(i * L, L)]
                return _

            lax.fori_loop(0, nv // L, body, None)
            pltpu.sync_copy(s_out.at[pl.ds(0, nv)], s_hbm.at[pl.ds(hb, nv)])
            pltpu.sync_copy(g_out.at[pl.ds(0, nv)], g_hbm.at[pl.ds(hb, nv)])

        for h in range(2):
            hb = obase + h * HS
            pltpu.sync_copy(et_hbm.at[pl.ds(hb, HS)], b_et)
            pltpu.sync_copy(dst_hbm.at[pl.ds(hb, HS)], b_dst)
            pltpu.sync_copy(src_hbm.at[pl.ds(hb, HS)], b_src)
            emit(HS, hb)
        tb = obase + 2 * HS
        pltpu.sync_copy(et_hbm.at[pl.ds(tb, TAIL)], b_et.at[pl.ds(0, TAIL)])
        pltpu.sync_copy(dst_hbm.at[pl.ds(tb, TAIL)], b_dst.at[pl.ds(0, TAIL)])
        pltpu.sync_copy(src_hbm.at[pl.ds(tb, TAIL)], b_src.at[pl.ds(0, TAIL)])
        emit(TAIL, tb)

    return k


def _sc_main(E, N, R, NC, NS):
    EPT = E // (NC * NS)
    CH = 80
    NCHUNK = EPT // CH
    RPT = (N // NS) // 8 * 8
    REM = N - RPT * NS
    mesh = plsc.VectorSubcoreMesh(core_axis_name="c", subcore_axis_name="s")

    @functools.partial(
        pl.kernel,
        mesh=mesh,
        out_type=jax.ShapeDtypeStruct((NC, N, 128), jnp.float32),
        scratch_types=[
            pltpu.VMEM((EPT,), jnp.int32),
            pltpu.VMEM((CH, 128), jnp.float32),
            pltpu.VMEM((CH, 128), jnp.float32),
            pltpu.VMEM((CH, 128), jnp.float32),
            pltpu.VMEM((3, CH), jnp.int32),
            pltpu.VMEM((3, CH), jnp.float32),
            pltpu.VMEM_SHARED((N, 128), jnp.float32),
            pltpu.SemaphoreType.DMA,
            pltpu.SemaphoreType.DMA,
            pltpu.SemaphoreType.DMA,
            pltpu.SemaphoreType.DMA,
            pltpu.SemaphoreType.DMA,
            pltpu.SemaphoreType.DMA,
        ],
        compiler_params=pltpu.CompilerParams(needs_layout_passes=False),
    )
    def k(
        xw_hbm,
        g_hbm,
        dst_hbm,
        s_hbm,
        out_hbm,
        pre_g,
        rows0,
        rows1,
        rows2,
        dst_ix,
        s_ix,
        acc_sp,
        gsem0,
        gsem1,
        gsem2,
        ssem0,
        ssem1,
        ssem2,
    ):
        cid = lax.axis_index("c")
        sid = lax.axis_index("s")
        wid = cid * NS + sid
        tbase = wid * EPT
        zeros16 = jnp.zeros((L,), jnp.float32)
        rows = (rows0, rows1, rows2)
        gsem = (gsem0, gsem1, gsem2)
        ssem = (ssem0, ssem1, ssem2)

        def zrow(i, _):
            for j in range(8):
                rows0[i, pl.ds(j * L, L)] = zeros16
            return _

        lax.fori_loop(0, CH, zrow, None)
        nfull = RPT // CH
        for t in range(nfull):
            pltpu.sync_copy(rows0, acc_sp.at[pl.ds(sid * RPT + t * CH, CH)])
        rem = RPT - nfull * CH
        if rem:
            pltpu.sync_copy(
                rows0.at[pl.ds(0, rem)],
                acc_sp.at[pl.ds(sid * RPT + nfull * CH, rem)],
            )
        if REM:

            @pl.when(sid == NS - 1)
            def _():
                pltpu.sync_copy(
                    rows0.at[pl.ds(0, REM)], acc_sp.at[pl.ds(RPT * NS, REM)]
                )

        plsc.subcore_barrier()

        pltpu.sync_copy(g_hbm.at[pl.ds(tbase, EPT)], pre_g)

        def issue(c, b):
            pltpu.async_copy(
                xw_hbm.at[pre_g.at[pl.ds(c * CH, CH)]], rows[b], gsem[b]
            )
            pltpu.async_copy(
                dst_hbm.at[pl.ds(tbase + c * CH, CH)], dst_ix.at[b], gsem[b]
            )
            pltpu.async_copy(
                s_hbm.at[pl.ds(tbase + c * CH, CH)], s_ix.at[b], gsem[b]
            )

        def wait_in(c, b):
            pltpu.make_async_copy(
                xw_hbm.at[pre_g.at[pl.ds(c * CH, CH)]], rows[b], gsem[b]
            ).wait()
            pltpu.make_async_copy(
                dst_hbm.at[pl.ds(tbase + c * CH, CH)], dst_ix.at[b], gsem[b]
            ).wait()
            pltpu.make_async_copy(
                s_hbm.at[pl.ds(tbase + c * CH, CH)], s_ix.at[b], gsem[b]
            ).wait()

        def drain_sc(b):
            pltpu.make_async_copy(
                rows[b], acc_sp.at[dst_ix.at[b]], ssem[b]
            ).wait()

        issue(0, 0)

        def slot(c, b):
            bn = (b + 1) % 3
            pl.when(c >= 2)(lambda: drain_sc(bn))

            @pl.when(c <= NCHUNK - 2)
            def _():
                issue(c + 1, bn)

            wait_in(c, b)

            def scale(kk, _):
                sk = plsc.load_gather(
                    s_ix.at[b], [jnp.full((L,), kk, jnp.int32)]
                )
                rb = rows[b]
                for j in range(8):
                    rb[kk, pl.ds(j * L, L)] = rb[kk, pl.ds(j * L, L)] * sk
                return _

            lax.fori_loop(0, CH, scale, None)
            pltpu.async_copy(rows[b], acc_sp.at[dst_ix.at[b]], ssem[b], add=True)

        @pl.loop(0, NCHUNK - 2, step=3)
        def _(c):
            for b in range(3):
                slot(c + b, b)

        slot(NCHUNK - 2, (NCHUNK - 2) % 3)
        slot(NCHUNK - 1, (NCHUNK - 1) % 3)
        drain_sc((NCHUNK - 2) % 3)
        drain_sc((NCHUNK - 1) % 3)
        plsc.subcore_barrier()

        pltpu.sync_copy(
            acc_sp.at[pl.ds(sid * RPT, RPT)],
            out_hbm.at[cid, pl.ds(sid * RPT, RPT)],
        )
        if REM:

            @pl.when(sid == NS - 1)
            def _():
                pltpu.sync_copy(
                    acc_sp.at[pl.ds(RPT * NS, REM)],
                    out_hbm.at[cid, pl.ds(RPT * NS, REM)],
                )

    return k


def kernel(x, edge_index, edge_type, W, W_root, bias):
    N, D = x.shape
    R = W.shape[0]
    E = edge_type.shape[0]
    NC, NS = 2, 16
    CNT = 81920
    assert D == 128 and R * N <= CNT and E == 320000 and N == 10000

    src = edge_index[0]
    dst = edge_index[1]

    s, g = _sc_prep(E, N, R, NC, NS, CNT)(edge_type, dst, src)
    xw = _tc_xw(x, W).reshape(R * N, D)
    acc = _sc_main(E, N, R, NC, NS)(xw, g, dst, s)
    return _tc_out(x, W_root, bias, acc)
